# Initial kernel scaffold; baseline (speedup 1.0000x reference)
import jax
import jax.numpy as jnp
from jax.experimental import pallas as pl


def kernel(x, Wg, W1, b1, W2, b2):
    raise NotImplementedError("write your pallas kernel here")


# speedup vs baseline: 3.2138x; 3.2138x over previous
import jax
import jax.numpy as jnp
from jax.experimental import pallas as pl
from jax.experimental.pallas import tpu as pltpu

D_MODEL = 768
D_FF = 3072
N_EXP = 8
TOP_K = 2
T = 2048
TILE_T = 512
N_TT = T // TILE_T


def _moe_dense_body(x_ref, wg_ref, w1_ref, b1_ref, w2_ref, b2_ref, out_ref):
    e = pl.program_id(0)
    t = pl.program_id(1)
    x = x_ref[...]
    logits = jax.lax.dot_general(x, wg_ref[...],
                                 (((1,), (1,)), ((), ())),
                                 preferred_element_type=jnp.float32)
    m1 = jnp.max(logits, axis=-1, keepdims=True)
    a1 = jnp.argmax(logits, axis=-1)
    masked = jnp.where(jax.lax.broadcasted_iota(jnp.int32, logits.shape, 1)
                       == a1[:, None], -jnp.inf, logits)
    m2 = jnp.max(masked, axis=-1, keepdims=True)
    a2 = jnp.argmax(masked, axis=-1)
    w_first = 1.0 / (1.0 + jnp.exp(m2 - m1))
    w_second = 1.0 - w_first
    w_e = (jnp.where(a1 == e, w_first[:, 0], 0.0)
           + jnp.where(a2 == e, w_second[:, 0], 0.0))

    h = jax.lax.dot_general(x, w1_ref[0], (((1,), (1,)), ((), ())),
                            preferred_element_type=jnp.float32)
    h = h + b1_ref[0]
    h = 0.5 * h * (1.0 + jax.lax.erf(h * 0.7071067811865476))
    y = jax.lax.dot_general(h, w2_ref[0], (((1,), (1,)), ((), ())),
                            preferred_element_type=jnp.float32)
    y = y + b2_ref[0]
    y = y * w_e[:, None]

    @pl.when(e == 0)
    def _():
        out_ref[pl.ds(t * TILE_T, TILE_T), :] = jnp.zeros((TILE_T, D_MODEL),
                                                          jnp.float32)
    out_ref[pl.ds(t * TILE_T, TILE_T), :] += y


def kernel(x, Wg, W1, b1, W2, b2):
    B, S, D = x.shape
    x_flat = x.reshape(-1, D)
    out = pl.pallas_call(
        _moe_dense_body,
        grid=(N_EXP, N_TT),
        in_specs=[
            pl.BlockSpec((TILE_T, D_MODEL), lambda e, t: (t, 0)),
            pl.BlockSpec((N_EXP, D_MODEL), lambda e, t: (0, 0)),
            pl.BlockSpec((1, D_FF, D_MODEL), lambda e, t: (e, 0, 0)),
            pl.BlockSpec((1, 1, D_FF), lambda e, t: (e, 0, 0)),
            pl.BlockSpec((1, D_MODEL, D_FF), lambda e, t: (e, 0, 0)),
            pl.BlockSpec((1, 1, D_MODEL), lambda e, t: (e, 0, 0)),
        ],
        out_specs=pl.BlockSpec((T, D_MODEL), lambda e, t: (0, 0)),
        out_shape=jax.ShapeDtypeStruct((T, D_MODEL), jnp.float32),
    )(x_flat, Wg, W1, b1.reshape(N_EXP, 1, D_FF), W2,
      b2.reshape(N_EXP, 1, D_MODEL))
    return out.reshape(B, S, D)
